# baseline (speedup 1.0000x reference)
import functools

import numpy as np
import jax
import jax.numpy as jnp
from jax import lax
from jax.experimental import pallas as pl
from jax.experimental.pallas import tpu as pltpu
from jax.experimental.pallas import tpu_sc as plsc

NN = 50000
NP = 51200
ER = 100000
EP = 102400
EROW = EP // 128
NR = 6
HID = 128
BN = 3200
NB = NP // BN

_NC, _NS = 2, 16
_STR = NP // _NS

_GELU_C = float(np.sqrt(2.0 / np.pi))
_f32 = jnp.float32
_i32 = jnp.int32


def _gelu(x):
    return 0.5 * x * (1.0 + jnp.tanh(_GELU_C * (x + 0.044715 * x * x * x)))


def _layernorm_skip(trans, h, alpha, g1, b1):
    res = trans * alpha + h * (1.0 - alpha)
    mu = jnp.mean(res, axis=-1, keepdims=True)
    var = jnp.mean((res - mu) ** 2, axis=-1, keepdims=True)
    return (res - mu) * jax.lax.rsqrt(var + 1e-5) * g1 + b1


def _d0_body(x_ref, wp, bp, wq, bq, wkr, bkr, wvm, bvm,
             h_ref, q_ref, kr_ref, vm0, vm1, vm2, vm3):
    h = jnp.dot(x_ref[...], wp[...], preferred_element_type=_f32) + bp[...]
    h_ref[...] = h
    q_ref[...] = jnp.dot(h, wq[...], preferred_element_type=_f32) + bq[...]
    kr_ref[...] = jnp.dot(h, wkr[...], preferred_element_type=_f32) + bkr[...]
    vm = jnp.dot(h, wvm[...], preferred_element_type=_f32) + bvm[...]
    vm0[...] = vm[:, 0:32]
    vm1[...] = vm[:, 32:64]
    vm2[...] = vm[:, 64:96]
    vm3[...] = vm[:, 96:128]


def _mid_body(u0, u1, u2, u3, den, h_ref, wa, ba, al, g1, b1,
              wq, bq, wkr, bkr, wvm, bvm,
              h_out, q_ref, kr_ref, vm0, vm1, vm2, vm3):
    u = jnp.concatenate([u0[...], u1[...], u2[...], u3[...]], axis=1)
    d = den[:, 0] + den[:, 1] + 1e-9
    agg = u / d[:, None]
    trans = jnp.dot(_gelu(agg), wa[...], preferred_element_type=_f32) + ba[...]
    hn = _layernorm_skip(trans, h_ref[...], al[0, 0], g1[...], b1[...])
    h_out[...] = hn
    q_ref[...] = jnp.dot(hn, wq[...], preferred_element_type=_f32) + bq[...]
    kr_ref[...] = jnp.dot(hn, wkr[...], preferred_element_type=_f32) + bkr[...]
    vm = jnp.dot(hn, wvm[...], preferred_element_type=_f32) + bvm[...]
    vm0[...] = vm[:, 0:32]
    vm1[...] = vm[:, 32:64]
    vm2[...] = vm[:, 64:96]
    vm3[...] = vm[:, 96:128]


def _last_body(u0, u1, u2, u3, den, h_ref, wa, ba, al, g1, b1, acc_ref):
    u = jnp.concatenate([u0[...], u1[...], u2[...], u3[...]], axis=1)
    d = den[:, 0] + den[:, 1] + 1e-9
    agg = u / d[:, None]
    trans = jnp.dot(_gelu(agg), wa[...], preferred_element_type=_f32) + ba[...]
    hn = _layernorm_skip(trans, h_ref[...], al[0, 0], g1[...], b1[...])
    rid = (jax.lax.broadcasted_iota(_i32, (BN, 1), 0)
           + pl.program_id(0) * BN)
    hn = jnp.where(rid < NN, hn, 0.0)
    part = jnp.sum(hn, axis=0, keepdims=True)

    @pl.when(pl.program_id(0) == 0)
    def _():
        acc_ref[...] = part

    @pl.when(pl.program_id(0) != 0)
    def _():
        acc_ref[...] += part


def _row_spec(w):
    return pl.BlockSpec((BN, w), lambda i: (i, 0))


def _full_spec(shape):
    return pl.BlockSpec(shape, lambda i: tuple(0 for _ in shape))


_W128 = _full_spec((128, 128))
_B128 = _full_spec((1, 128))
_SCAL = _full_spec((1, 1))
_DEN = pl.BlockSpec((BN, 2), lambda i: (i, 0))


def _d0_call(x, wp, bp, wq, bq, wkr, bkr, wvm, bvm):
    outs = (
        jax.ShapeDtypeStruct((NP, 128), _f32),
        jax.ShapeDtypeStruct((NP, 128), _f32),
        jax.ShapeDtypeStruct((NP, 128), _f32),
    ) + tuple(jax.ShapeDtypeStruct((NP, 32), _f32) for _ in range(4))
    return pl.pallas_call(
        _d0_body,
        grid=(NB,),
        in_specs=[_row_spec(128), _W128, _B128, _W128, _B128, _W128, _B128,
                  _W128, _B128],
        out_specs=(_row_spec(128), _row_spec(128), _row_spec(128),
                   _row_spec(32), _row_spec(32), _row_spec(32), _row_spec(32)),
        out_shape=outs,
    )(x, wp, bp, wq, bq, wkr, bkr, wvm, bvm)


def _mid_call(u4, den, h, wa, ba, al, g1, b1, wq, bq, wkr, bkr, wvm, bvm):
    outs = (
        jax.ShapeDtypeStruct((NP, 128), _f32),
        jax.ShapeDtypeStruct((NP, 128), _f32),
        jax.ShapeDtypeStruct((NP, 128), _f32),
    ) + tuple(jax.ShapeDtypeStruct((NP, 32), _f32) for _ in range(4))
    return pl.pallas_call(
        _mid_body,
        grid=(NB,),
        in_specs=[_row_spec(32)] * 4 + [_DEN, _row_spec(128),
                  _W128, _B128, _SCAL, _B128, _B128,
                  _W128, _B128, _W128, _B128, _W128, _B128],
        out_specs=(_row_spec(128), _row_spec(128), _row_spec(128),
                   _row_spec(32), _row_spec(32), _row_spec(32), _row_spec(32)),
        out_shape=outs,
    )(*u4, den, h, wa, ba, al, g1, b1, wq, bq, wkr, bkr, wvm, bvm)


def _last_call(u4, den, h, wa, ba, al, g1, b1):
    return pl.pallas_call(
        _last_body,
        grid=(NB,),
        in_specs=[_row_spec(32)] * 4 + [_DEN, _row_spec(128),
                  _W128, _B128, _SCAL, _B128, _B128],
        out_specs=pl.BlockSpec((1, 128), lambda i: (0, 0)),
        out_shape=jax.ShapeDtypeStruct((1, 128), _f32),
    )(*u4, den, h, wa, ba, al, g1, b1)


def _sc_mesh():
    return plsc.VectorSubcoreMesh(core_axis_name="c", subcore_axis_name="s",
                                  num_cores=_NC, num_subcores=_NS)


def _att_body(q_hbm, kr_hbm, src_hbm, dst_hbm, z1_hbm,
              ae_hbm, den0_hbm, den1_hbm,
              srcb, dstb, qrows, krrows, aebuf, den_sh, sem):
    ci = lax.axis_index("c")
    si = lax.axis_index("s")
    t = ci * _NS + si
    e0 = t * 3200
    pltpu.sync_copy(z1_hbm, den_sh.at[pl.ds(si * _STR, _STR)])
    pltpu.sync_copy(src_hbm.at[pl.ds(e0, 3200)], srcb)

    def ldrow(j, carry):
        pltpu.sync_copy(dst_hbm.at[pl.ds(e0 + j * 128, 128)], dstb.at[j])
        return carry

    lax.fori_loop(0, 25, ldrow, 0)
    plsc.subcore_barrier()
    iota = lax.iota(_i32, 16)

    def block(j, carry):
        cp1 = pltpu.async_copy(q_hbm.at[dstb.at[j]], qrows, sem)
        cp2 = pltpu.async_copy(kr_hbm.at[srcb.at[pl.ds(j * 128, 128)]],
                               krrows, sem)
        cp1.wait()
        cp2.wait()
        base = e0 + j * 128
        for g in range(8):
            rowv = g * 16 + iota

            def dstep(dd, acc):
                colv = jnp.zeros((16,), _i32) + dd
                qv = plsc.load_gather(qrows, [rowv, colv])
                kv = plsc.load_gather(krrows, [rowv, colv])
                return acc + qv * kv

            acc = lax.fori_loop(0, 128, dstep, jnp.zeros((16,), _f32),
                                unroll=8)
            ae = jnp.exp(acc)
            gid = base + g * 16 + iota
            ae = jnp.where(gid < ER, ae, 0.0)
            aebuf[pl.ds(j * 128 + g * 16, 16)] = ae
        pltpu.sync_copy(aebuf.at[pl.ds(j * 128, 128)],
                        den_sh.at[dstb.at[j]], add=True)
        return carry

    lax.fori_loop(0, 25, block, 0)
    pltpu.sync_copy(aebuf, ae_hbm.at[pl.ds(e0, 3200)])
    plsc.subcore_barrier()

    @pl.when(ci == 0)
    def _():
        pltpu.sync_copy(den_sh.at[pl.ds(si * _STR, _STR)],
                        den0_hbm.at[pl.ds(si * _STR, _STR)])

    @pl.when(ci == 1)
    def _():
        pltpu.sync_copy(den_sh.at[pl.ds(si * _STR, _STR)],
                        den1_hbm.at[pl.ds(si * _STR, _STR)])


def _att_call(q, kr, src3, dst3, z1):
    out_type = (jax.ShapeDtypeStruct((EP,), _f32),
                jax.ShapeDtypeStruct((NP,), _f32),
                jax.ShapeDtypeStruct((NP,), _f32))
    scratch = [
        pltpu.VMEM((3200,), _i32),
        pltpu.VMEM((25, 128), _i32),
        pltpu.VMEM((128, 128), _f32),
        pltpu.VMEM((128, 128), _f32),
        pltpu.VMEM((3200,), _f32),
        pltpu.VMEM_SHARED((NP,), _f32),
        pltpu.SemaphoreType.DMA,
    ]
    f = functools.partial(
        pl.kernel, out_type=out_type, mesh=_sc_mesh(),
        scratch_types=scratch,
        compiler_params=pltpu.CompilerParams(needs_layout_passes=False),
    )(_att_body)
    return f(q, kr, src3, dst3, z1)


def _u_body(vm0, vm1, vm2, vm3, ae_hbm, src_hbm, dst_hbm, z2_hbm,
            u0, u1, u2, u3,
            srcb, dstb, aeb, vrows, u_sh, sem):
    ci = lax.axis_index("c")
    si = lax.axis_index("s")
    e0 = si * 6400
    pltpu.sync_copy(src_hbm.at[pl.ds(e0, 6400)], srcb)
    pltpu.sync_copy(ae_hbm.at[pl.ds(e0, 6400)], aeb)

    def ldrow(j, carry):
        pltpu.sync_copy(dst_hbm.at[pl.ds(e0 + j * 128, 128)], dstb.at[j])
        return carry

    lax.fori_loop(0, 50, ldrow, 0)
    iota = lax.iota(_i32, 16)
    vms = (vm0, vm1, vm2, vm3)
    us = (u0, u1, u2, u3)
    for p in range(2):
        pltpu.sync_copy(z2_hbm, u_sh.at[pl.ds(si * _STR, _STR)])
        plsc.subcore_barrier()
        for cival in range(2):
            chunk = 2 * p + cival

            @pl.when(ci == cival)
            def _(chunk=chunk):
                vmr = vms[chunk]

                def block(j, carry):
                    idx = srcb.at[pl.ds(j * 128, 128)]
                    pltpu.async_copy(vmr.at[idx], vrows, sem).wait()
                    for g in range(8):
                        rowv = g * 16 + iota
                        aev = aeb[pl.ds(j * 128 + g * 16, 16)]

                        def dstep(dd, cc):
                            colv = jnp.zeros((16,), _i32) + dd
                            x = plsc.load_gather(vrows, [rowv, colv])
                            plsc.store_scatter(vrows, [rowv, colv], x * aev)
                            return cc

                        lax.fori_loop(0, 32, dstep, 0, unroll=8)
                    pltpu.sync_copy(vrows, u_sh.at[dstb.at[j]], add=True)
                    return carry

                lax.fori_loop(0, 50, block, 0)
        plsc.subcore_barrier()
        for cival in range(2):
            chunk = 2 * p + cival

            @pl.when(ci == cival)
            def _(chunk=chunk):
                pltpu.sync_copy(u_sh.at[pl.ds(si * _STR, _STR)],
                                us[chunk].at[pl.ds(si * _STR, _STR)])
        plsc.subcore_barrier()


def _u_call(vm4, ae3u, src3u, dst3u, z2):
    out_type = tuple(jax.ShapeDtypeStruct((NP, 32), _f32) for _ in range(4))
    scratch = [
        pltpu.VMEM((6400,), _i32),
        pltpu.VMEM((50, 128), _i32),
        pltpu.VMEM((6400,), _f32),
        pltpu.VMEM((128, 32), _f32),
        pltpu.VMEM_SHARED((NP, 32), _f32),
        pltpu.SemaphoreType.DMA,
    ]
    f = functools.partial(
        pl.kernel, out_type=out_type, mesh=_sc_mesh(),
        scratch_types=scratch,
        compiler_params=pltpu.CompilerParams(needs_layout_passes=False,
                                             use_tc_tiling_on_sc=False),
    )(_u_body)
    return f(*vm4, ae3u, src3u, dst3u, z2)


def kernel(x, edge_index, edge_weight, params):
    lys = params['layers']
    n_layers = len(lys)
    conv_w = []
    for l in range(n_layers):
        lp = lys[l]
        for i in range(NR):
            s = lp['pri'][i, 0] / np.sqrt(np.float32(HID))
            conv_w.append(dict(
                wq=lp['Wq'], bq=lp['bq'][None, :],
                wkr=(lp['Wk'] @ lp['ratt'][i, 0]) * s,
                bkr=(lp['bk'] @ lp['ratt'][i, 0])[None, :] * s,
                wvm=lp['Wv'] @ lp['rmsg'][i, 0],
                bvm=(lp['bv'] @ lp['rmsg'][i, 0])[None, :],
                wa=lp['Wa'], ba=lp['ba'][None, :],
                al=jax.nn.sigmoid(lp['skip']).reshape(1, 1),
                g1=lp['g1'][None, :], b1=lp['b1'][None, :]))

    srcs, dsts = [], []
    for i in range(NR):
        srcs.append(jnp.pad(edge_index[i, 0], (0, EP - ER)))
        dsts.append(jnp.pad(edge_index[i, 1], (0, EP - ER)))
    z1 = jnp.zeros((_STR,), _f32)
    z2 = jnp.zeros((_STR, 32), _f32)

    x_p = jnp.pad(x, ((0, NP - NN), (0, 0)))
    w0 = conv_w[0]
    h, q, kr, *vm4 = _d0_call(
        x_p, params['Wp'], params['bp'][None, :],
        w0['wq'], w0['bq'], w0['wkr'], w0['bkr'], w0['wvm'], w0['bvm'])
    vm4 = tuple(vm4)

    n_conv = n_layers * NR
    for j in range(n_conv):
        ae1, den0, den1 = _att_call(q, kr, srcs[j % NR], dsts[j % NR], z1)
        u4 = _u_call(vm4, ae1, srcs[j % NR], dsts[j % NR], z2)
        den_t = jnp.stack([den0, den1], axis=1)
        w = conv_w[j]
        if j + 1 < n_conv:
            wn = conv_w[j + 1]
            h, q, kr, *vm4 = _mid_call(
                u4, den_t, h, w['wa'], w['ba'], w['al'], w['g1'], w['b1'],
                wn['wq'], wn['bq'], wn['wkr'], wn['bkr'], wn['wvm'], wn['bvm'])
            vm4 = tuple(vm4)
        else:
            hsum = _last_call(u4, den_t, h, w['wa'], w['ba'], w['al'],
                              w['g1'], w['b1'])

    hg = hsum[0] / np.float32(NN)
    logits = (hg @ params['Wc'] + params['bc']).squeeze(-1)
    return logits


# speedup vs baseline: 1.2499x; 1.2499x over previous
import functools

import numpy as np
import jax
import jax.numpy as jnp
from jax import lax
from jax.experimental import pallas as pl
from jax.experimental.pallas import tpu as pltpu
from jax.experimental.pallas import tpu_sc as plsc

NN = 50000
NP = 51200
ER = 100000
EP = 102400
EROW = EP // 128
NR = 6
HID = 128
BN = 3200
NB = NP // BN

_NC, _NS = 2, 16
_STR = NP // _NS

_GELU_C = float(np.sqrt(2.0 / np.pi))
_f32 = jnp.float32
_i32 = jnp.int32


def _gelu(x):
    return 0.5 * x * (1.0 + jnp.tanh(_GELU_C * (x + 0.044715 * x * x * x)))


def _layernorm_skip(trans, h, alpha, g1, b1):
    res = trans * alpha + h * (1.0 - alpha)
    mu = jnp.mean(res, axis=-1, keepdims=True)
    var = jnp.mean((res - mu) ** 2, axis=-1, keepdims=True)
    return (res - mu) * jax.lax.rsqrt(var + 1e-5) * g1 + b1


def _d0_body(x_ref, wp, bp, wq, bq, wkr, bkr, wvm, bvm,
             h_ref, q_ref, kr_ref, vm0, vm1, vm2, vm3):
    h = jnp.dot(x_ref[...], wp[...], preferred_element_type=_f32) + bp[...]
    h_ref[...] = h
    q_ref[...] = jnp.dot(h, wq[...], preferred_element_type=_f32) + bq[...]
    kr_ref[...] = jnp.dot(h, wkr[...], preferred_element_type=_f32) + bkr[...]
    vm = jnp.dot(h, wvm[...], preferred_element_type=_f32) + bvm[...]
    vm0[...] = vm[:, 0:32]
    vm1[...] = vm[:, 32:64]
    vm2[...] = vm[:, 64:96]
    vm3[...] = vm[:, 96:128]


def _mid_body(u0, u1, u2, u3, den, h_ref, wa, ba, al, g1, b1,
              wq, bq, wkr, bkr, wvm, bvm,
              h_out, q_ref, kr_ref, vm0, vm1, vm2, vm3):
    u = jnp.concatenate([u0[...], u1[...], u2[...], u3[...]], axis=1)
    d = den[:, 0] + den[:, 1] + 1e-9
    agg = u / d[:, None]
    trans = jnp.dot(_gelu(agg), wa[...], preferred_element_type=_f32) + ba[...]
    hn = _layernorm_skip(trans, h_ref[...], al[0, 0], g1[...], b1[...])
    h_out[...] = hn
    q_ref[...] = jnp.dot(hn, wq[...], preferred_element_type=_f32) + bq[...]
    kr_ref[...] = jnp.dot(hn, wkr[...], preferred_element_type=_f32) + bkr[...]
    vm = jnp.dot(hn, wvm[...], preferred_element_type=_f32) + bvm[...]
    vm0[...] = vm[:, 0:32]
    vm1[...] = vm[:, 32:64]
    vm2[...] = vm[:, 64:96]
    vm3[...] = vm[:, 96:128]


def _last_body(u0, u1, u2, u3, den, h_ref, wa, ba, al, g1, b1, acc_ref):
    u = jnp.concatenate([u0[...], u1[...], u2[...], u3[...]], axis=1)
    d = den[:, 0] + den[:, 1] + 1e-9
    agg = u / d[:, None]
    trans = jnp.dot(_gelu(agg), wa[...], preferred_element_type=_f32) + ba[...]
    hn = _layernorm_skip(trans, h_ref[...], al[0, 0], g1[...], b1[...])
    rid = (jax.lax.broadcasted_iota(_i32, (BN, 1), 0)
           + pl.program_id(0) * BN)
    hn = jnp.where(rid < NN, hn, 0.0)
    part = jnp.sum(hn, axis=0, keepdims=True)

    @pl.when(pl.program_id(0) == 0)
    def _():
        acc_ref[...] = part

    @pl.when(pl.program_id(0) != 0)
    def _():
        acc_ref[...] += part


def _row_spec(w):
    return pl.BlockSpec((BN, w), lambda i: (i, 0))


def _full_spec(shape):
    return pl.BlockSpec(shape, lambda i: tuple(0 for _ in shape))


_W128 = _full_spec((128, 128))
_B128 = _full_spec((1, 128))
_SCAL = _full_spec((1, 1))
_DEN = pl.BlockSpec((BN, 2), lambda i: (i, 0))


def _d0_call(x, wp, bp, wq, bq, wkr, bkr, wvm, bvm):
    outs = (
        jax.ShapeDtypeStruct((NP, 128), _f32),
        jax.ShapeDtypeStruct((NP, 128), _f32),
        jax.ShapeDtypeStruct((NP, 128), _f32),
    ) + tuple(jax.ShapeDtypeStruct((NP, 32), _f32) for _ in range(4))
    return pl.pallas_call(
        _d0_body,
        grid=(NB,),
        in_specs=[_row_spec(128), _W128, _B128, _W128, _B128, _W128, _B128,
                  _W128, _B128],
        out_specs=(_row_spec(128), _row_spec(128), _row_spec(128),
                   _row_spec(32), _row_spec(32), _row_spec(32), _row_spec(32)),
        out_shape=outs,
    )(x, wp, bp, wq, bq, wkr, bkr, wvm, bvm)


def _mid_call(u4, den, h, wa, ba, al, g1, b1, wq, bq, wkr, bkr, wvm, bvm):
    outs = (
        jax.ShapeDtypeStruct((NP, 128), _f32),
        jax.ShapeDtypeStruct((NP, 128), _f32),
        jax.ShapeDtypeStruct((NP, 128), _f32),
    ) + tuple(jax.ShapeDtypeStruct((NP, 32), _f32) for _ in range(4))
    return pl.pallas_call(
        _mid_body,
        grid=(NB,),
        in_specs=[_row_spec(32)] * 4 + [_DEN, _row_spec(128),
                  _W128, _B128, _SCAL, _B128, _B128,
                  _W128, _B128, _W128, _B128, _W128, _B128],
        out_specs=(_row_spec(128), _row_spec(128), _row_spec(128),
                   _row_spec(32), _row_spec(32), _row_spec(32), _row_spec(32)),
        out_shape=outs,
    )(*u4, den, h, wa, ba, al, g1, b1, wq, bq, wkr, bkr, wvm, bvm)


def _last_call(u4, den, h, wa, ba, al, g1, b1):
    return pl.pallas_call(
        _last_body,
        grid=(NB,),
        in_specs=[_row_spec(32)] * 4 + [_DEN, _row_spec(128),
                  _W128, _B128, _SCAL, _B128, _B128],
        out_specs=pl.BlockSpec((1, 128), lambda i: (0, 0)),
        out_shape=jax.ShapeDtypeStruct((1, 128), _f32),
    )(*u4, den, h, wa, ba, al, g1, b1)


def _sc_mesh():
    return plsc.VectorSubcoreMesh(core_axis_name="c", subcore_axis_name="s",
                                  num_cores=_NC, num_subcores=_NS)


def _att_body(q_hbm, kr_hbm, src_hbm, dst_hbm, z1_hbm,
              ae_hbm, den0_hbm, den1_hbm,
              srcb, dstb, qrowsA, krrowsA, qrowsB, krrowsB,
              aebuf, den_sh, semA, semB):
    ci = lax.axis_index("c")
    si = lax.axis_index("s")
    t = ci * _NS + si
    e0 = t * 3200
    pltpu.sync_copy(z1_hbm, den_sh.at[pl.ds(si * _STR, _STR)])
    pltpu.sync_copy(src_hbm.at[pl.ds(e0, 3200)], srcb)

    def ldrow(j, carry):
        pltpu.sync_copy(dst_hbm.at[pl.ds(e0 + j * 128, 128)], dstb.at[j])
        return carry

    lax.fori_loop(0, 25, ldrow, 0)
    plsc.subcore_barrier()
    iota = lax.iota(_i32, 16)

    def start(j, qr, krr, sm):
        pltpu.async_copy(q_hbm.at[dstb.at[j]], qr, sm)
        pltpu.async_copy(kr_hbm.at[srcb.at[pl.ds(j * 128, 128)]], krr, sm)

    def wait(j, qr, krr, sm):
        pltpu.make_async_copy(q_hbm.at[dstb.at[j]], qr, sm).wait()
        pltpu.make_async_copy(kr_hbm.at[srcb.at[pl.ds(j * 128, 128)]],
                              krr, sm).wait()

    def compute(j, qr, krr):
        base = e0 + j * 128
        for g in range(8):
            rowv = g * 16 + iota

            def dstep(dd, acc):
                colv = jnp.zeros((16,), _i32) + dd
                qv = plsc.load_gather(qr, [rowv, colv])
                kv = plsc.load_gather(krr, [rowv, colv])
                return acc + qv * kv

            acc = lax.fori_loop(0, 128, dstep, jnp.zeros((16,), _f32),
                                unroll=8)
            ae = jnp.exp(acc)
            gid = base + g * 16 + iota
            ae = jnp.where(gid < ER, ae, 0.0)
            aebuf[pl.ds(j * 128 + g * 16, 16)] = ae
        pltpu.sync_copy(aebuf.at[pl.ds(j * 128, 128)],
                        den_sh.at[dstb.at[j]], add=True)

    start(0, qrowsA, krrowsA, semA)

    def block2(k, carry):
        j = 2 * k
        start(j + 1, qrowsB, krrowsB, semB)
        wait(j, qrowsA, krrowsA, semA)
        compute(j, qrowsA, krrowsA)
        start(j + 2, qrowsA, krrowsA, semA)
        wait(j + 1, qrowsB, krrowsB, semB)
        compute(j + 1, qrowsB, krrowsB)
        return carry

    lax.fori_loop(0, 12, block2, 0)
    wait(24, qrowsA, krrowsA, semA)
    compute(24, qrowsA, krrowsA)
    pltpu.sync_copy(aebuf, ae_hbm.at[pl.ds(e0, 3200)])
    plsc.subcore_barrier()

    @pl.when(ci == 0)
    def _():
        pltpu.sync_copy(den_sh.at[pl.ds(si * _STR, _STR)],
                        den0_hbm.at[pl.ds(si * _STR, _STR)])

    @pl.when(ci == 1)
    def _():
        pltpu.sync_copy(den_sh.at[pl.ds(si * _STR, _STR)],
                        den1_hbm.at[pl.ds(si * _STR, _STR)])


def _att_call(q, kr, src3, dst3, z1):
    out_type = (jax.ShapeDtypeStruct((EP,), _f32),
                jax.ShapeDtypeStruct((NP,), _f32),
                jax.ShapeDtypeStruct((NP,), _f32))
    scratch = [
        pltpu.VMEM((3200,), _i32),
        pltpu.VMEM((25, 128), _i32),
        pltpu.VMEM((128, 128), _f32),
        pltpu.VMEM((128, 128), _f32),
        pltpu.VMEM((128, 128), _f32),
        pltpu.VMEM((128, 128), _f32),
        pltpu.VMEM((3200,), _f32),
        pltpu.VMEM_SHARED((NP,), _f32),
        pltpu.SemaphoreType.DMA,
        pltpu.SemaphoreType.DMA,
    ]
    f = functools.partial(
        pl.kernel, out_type=out_type, mesh=_sc_mesh(),
        scratch_types=scratch,
        compiler_params=pltpu.CompilerParams(needs_layout_passes=False),
    )(_att_body)
    return f(q, kr, src3, dst3, z1)


def _u_body(vm0, vm1, vm2, vm3, ae_hbm, src_hbm, dst_hbm, z2_hbm,
            u0, u1, u2, u3,
            srcb, dstb, aeb, vrowsA, vrowsB, u_sh, semA, semB):
    ci = lax.axis_index("c")
    si = lax.axis_index("s")
    e0 = si * 6400
    pltpu.sync_copy(src_hbm.at[pl.ds(e0, 6400)], srcb)
    pltpu.sync_copy(ae_hbm.at[pl.ds(e0, 6400)], aeb)

    def ldrow(j, carry):
        pltpu.sync_copy(dst_hbm.at[pl.ds(e0 + j * 128, 128)], dstb.at[j])
        return carry

    lax.fori_loop(0, 50, ldrow, 0)
    iota = lax.iota(_i32, 16)
    vms = (vm0, vm1, vm2, vm3)
    us = (u0, u1, u2, u3)
    for p in range(2):
        pltpu.sync_copy(z2_hbm, u_sh.at[pl.ds(si * _STR, _STR)])
        plsc.subcore_barrier()
        for cival in range(2):
            chunk = 2 * p + cival

            @pl.when(ci == cival)
            def _(chunk=chunk):
                vmr = vms[chunk]

                def start(j, vr, sm):
                    pltpu.async_copy(vmr.at[srcb.at[pl.ds(j * 128, 128)]],
                                     vr, sm)

                def compute(j, vr, sm):
                    pltpu.make_async_copy(
                        vmr.at[srcb.at[pl.ds(j * 128, 128)]], vr, sm).wait()
                    for g in range(8):
                        rowv = g * 16 + iota
                        aev = aeb[pl.ds(j * 128 + g * 16, 16)]

                        def dstep(dd, cc):
                            colv = jnp.zeros((16,), _i32) + dd
                            x = plsc.load_gather(vr, [rowv, colv])
                            plsc.store_scatter(vr, [rowv, colv], x * aev)
                            return cc

                        lax.fori_loop(0, 32, dstep, 0, unroll=8)
                    pltpu.sync_copy(vr, u_sh.at[dstb.at[j]], add=True)

                start(0, vrowsA, semA)

                def block2(k, carry):
                    j = 2 * k
                    start(j + 1, vrowsB, semB)
                    compute(j, vrowsA, semA)

                    @pl.when(j + 2 < 50)
                    def _():
                        start(j + 2, vrowsA, semA)

                    compute(j + 1, vrowsB, semB)
                    return carry

                lax.fori_loop(0, 25, block2, 0)
        plsc.subcore_barrier()
        for cival in range(2):
            chunk = 2 * p + cival

            @pl.when(ci == cival)
            def _(chunk=chunk):
                pltpu.sync_copy(u_sh.at[pl.ds(si * _STR, _STR)],
                                us[chunk].at[pl.ds(si * _STR, _STR)])
        plsc.subcore_barrier()


def _u_call(vm4, ae3u, src3u, dst3u, z2):
    out_type = tuple(jax.ShapeDtypeStruct((NP, 32), _f32) for _ in range(4))
    scratch = [
        pltpu.VMEM((6400,), _i32),
        pltpu.VMEM((50, 128), _i32),
        pltpu.VMEM((6400,), _f32),
        pltpu.VMEM((128, 32), _f32),
        pltpu.VMEM((128, 32), _f32),
        pltpu.VMEM_SHARED((NP, 32), _f32),
        pltpu.SemaphoreType.DMA,
        pltpu.SemaphoreType.DMA,
    ]
    f = functools.partial(
        pl.kernel, out_type=out_type, mesh=_sc_mesh(),
        scratch_types=scratch,
        compiler_params=pltpu.CompilerParams(needs_layout_passes=False,
                                             use_tc_tiling_on_sc=False),
    )(_u_body)
    return f(*vm4, ae3u, src3u, dst3u, z2)


def kernel(x, edge_index, edge_weight, params):
    lys = params['layers']
    n_layers = len(lys)
    conv_w = []
    for l in range(n_layers):
        lp = lys[l]
        for i in range(NR):
            s = lp['pri'][i, 0] / np.sqrt(np.float32(HID))
            conv_w.append(dict(
                wq=lp['Wq'], bq=lp['bq'][None, :],
                wkr=(lp['Wk'] @ lp['ratt'][i, 0]) * s,
                bkr=(lp['bk'] @ lp['ratt'][i, 0])[None, :] * s,
                wvm=lp['Wv'] @ lp['rmsg'][i, 0],
                bvm=(lp['bv'] @ lp['rmsg'][i, 0])[None, :],
                wa=lp['Wa'], ba=lp['ba'][None, :],
                al=jax.nn.sigmoid(lp['skip']).reshape(1, 1),
                g1=lp['g1'][None, :], b1=lp['b1'][None, :]))

    srcs, dsts = [], []
    for i in range(NR):
        srcs.append(jnp.pad(edge_index[i, 0], (0, EP - ER)))
        dsts.append(jnp.pad(edge_index[i, 1], (0, EP - ER)))
    z1 = jnp.zeros((_STR,), _f32)
    z2 = jnp.zeros((_STR, 32), _f32)

    x_p = jnp.pad(x, ((0, NP - NN), (0, 0)))
    w0 = conv_w[0]
    h, q, kr, *vm4 = _d0_call(
        x_p, params['Wp'], params['bp'][None, :],
        w0['wq'], w0['bq'], w0['wkr'], w0['bkr'], w0['wvm'], w0['bvm'])
    vm4 = tuple(vm4)

    n_conv = n_layers * NR
    for j in range(n_conv):
        ae1, den0, den1 = _att_call(q, kr, srcs[j % NR], dsts[j % NR], z1)
        u4 = _u_call(vm4, ae1, srcs[j % NR], dsts[j % NR], z2)
        den_t = jnp.stack([den0, den1], axis=1)
        w = conv_w[j]
        if j + 1 < n_conv:
            wn = conv_w[j + 1]
            h, q, kr, *vm4 = _mid_call(
                u4, den_t, h, w['wa'], w['ba'], w['al'], w['g1'], w['b1'],
                wn['wq'], wn['bq'], wn['wkr'], wn['bkr'], wn['wvm'], wn['bvm'])
            vm4 = tuple(vm4)
        else:
            hsum = _last_call(u4, den_t, h, w['wa'], w['ba'], w['al'],
                              w['g1'], w['b1'])

    hg = hsum[0] / np.float32(NN)
    logits = (hg @ params['Wc'] + params['bc']).squeeze(-1)
    return logits


# speedup vs baseline: 1.2802x; 1.0243x over previous
import functools

import numpy as np
import jax
import jax.numpy as jnp
from jax import lax
from jax.experimental import pallas as pl
from jax.experimental.pallas import tpu as pltpu
from jax.experimental.pallas import tpu_sc as plsc

NN = 50000
NP = 51200
ER = 100000
EP = 102400
EROW = EP // 128
NR = 6
HID = 128
BN = 3200
NB = NP // BN

_NC, _NS = 2, 16
_STR = NP // _NS

_GELU_C = float(np.sqrt(2.0 / np.pi))
_f32 = jnp.float32
_i32 = jnp.int32


def _gelu(x):
    return 0.5 * x * (1.0 + jnp.tanh(_GELU_C * (x + 0.044715 * x * x * x)))


def _layernorm_skip(trans, h, alpha, g1, b1):
    res = trans * alpha + h * (1.0 - alpha)
    mu = jnp.mean(res, axis=-1, keepdims=True)
    var = jnp.mean((res - mu) ** 2, axis=-1, keepdims=True)
    return (res - mu) * jax.lax.rsqrt(var + 1e-5) * g1 + b1


def _d0_body(x_ref, wp, bp, wq, bq, wkr, bkr, wvm, bvm,
             h_ref, q_ref, kr_ref, vm0, vm1, vm2, vm3):
    h = jnp.dot(x_ref[...], wp[...], preferred_element_type=_f32) + bp[...]
    h_ref[...] = h
    q_ref[...] = jnp.dot(h, wq[...], preferred_element_type=_f32) + bq[...]
    kr_ref[...] = jnp.dot(h, wkr[...], preferred_element_type=_f32) + bkr[...]
    vm = jnp.dot(h, wvm[...], preferred_element_type=_f32) + bvm[...]
    vm0[...] = vm[:, 0:32]
    vm1[...] = vm[:, 32:64]
    vm2[...] = vm[:, 64:96]
    vm3[...] = vm[:, 96:128]


def _mid_body(u0, u1, u2, u3, den, h_ref, wa, ba, al, g1, b1,
              wq, bq, wkr, bkr, wvm, bvm,
              h_out, q_ref, kr_ref, vm0, vm1, vm2, vm3):
    u = jnp.concatenate([u0[...], u1[...], u2[...], u3[...]], axis=1)
    d = den[:, 0] + den[:, 1] + 1e-9
    agg = u / d[:, None]
    trans = jnp.dot(_gelu(agg), wa[...], preferred_element_type=_f32) + ba[...]
    hn = _layernorm_skip(trans, h_ref[...], al[0, 0], g1[...], b1[...])
    h_out[...] = hn
    q_ref[...] = jnp.dot(hn, wq[...], preferred_element_type=_f32) + bq[...]
    kr_ref[...] = jnp.dot(hn, wkr[...], preferred_element_type=_f32) + bkr[...]
    vm = jnp.dot(hn, wvm[...], preferred_element_type=_f32) + bvm[...]
    vm0[...] = vm[:, 0:32]
    vm1[...] = vm[:, 32:64]
    vm2[...] = vm[:, 64:96]
    vm3[...] = vm[:, 96:128]


def _last_body(u0, u1, u2, u3, den, h_ref, wa, ba, al, g1, b1, acc_ref):
    u = jnp.concatenate([u0[...], u1[...], u2[...], u3[...]], axis=1)
    d = den[:, 0] + den[:, 1] + 1e-9
    agg = u / d[:, None]
    trans = jnp.dot(_gelu(agg), wa[...], preferred_element_type=_f32) + ba[...]
    hn = _layernorm_skip(trans, h_ref[...], al[0, 0], g1[...], b1[...])
    rid = (jax.lax.broadcasted_iota(_i32, (BN, 1), 0)
           + pl.program_id(0) * BN)
    hn = jnp.where(rid < NN, hn, 0.0)
    part = jnp.sum(hn, axis=0, keepdims=True)

    @pl.when(pl.program_id(0) == 0)
    def _():
        acc_ref[...] = part

    @pl.when(pl.program_id(0) != 0)
    def _():
        acc_ref[...] += part


def _row_spec(w):
    return pl.BlockSpec((BN, w), lambda i: (i, 0))


def _full_spec(shape):
    return pl.BlockSpec(shape, lambda i: tuple(0 for _ in shape))


_W128 = _full_spec((128, 128))
_B128 = _full_spec((1, 128))
_SCAL = _full_spec((1, 1))
_DEN = pl.BlockSpec((BN, 2), lambda i: (i, 0))


def _d0_call(x, wp, bp, wq, bq, wkr, bkr, wvm, bvm):
    outs = (
        jax.ShapeDtypeStruct((NP, 128), _f32),
        jax.ShapeDtypeStruct((NP, 128), _f32),
        jax.ShapeDtypeStruct((NP, 128), _f32),
    ) + tuple(jax.ShapeDtypeStruct((NP, 32), _f32) for _ in range(4))
    return pl.pallas_call(
        _d0_body,
        grid=(NB,),
        in_specs=[_row_spec(128), _W128, _B128, _W128, _B128, _W128, _B128,
                  _W128, _B128],
        out_specs=(_row_spec(128), _row_spec(128), _row_spec(128),
                   _row_spec(32), _row_spec(32), _row_spec(32), _row_spec(32)),
        out_shape=outs,
    )(x, wp, bp, wq, bq, wkr, bkr, wvm, bvm)


def _mid_call(u4, den, h, wa, ba, al, g1, b1, wq, bq, wkr, bkr, wvm, bvm):
    outs = (
        jax.ShapeDtypeStruct((NP, 128), _f32),
        jax.ShapeDtypeStruct((NP, 128), _f32),
        jax.ShapeDtypeStruct((NP, 128), _f32),
    ) + tuple(jax.ShapeDtypeStruct((NP, 32), _f32) for _ in range(4))
    return pl.pallas_call(
        _mid_body,
        grid=(NB,),
        in_specs=[_row_spec(32)] * 4 + [_DEN, _row_spec(128),
                  _W128, _B128, _SCAL, _B128, _B128,
                  _W128, _B128, _W128, _B128, _W128, _B128],
        out_specs=(_row_spec(128), _row_spec(128), _row_spec(128),
                   _row_spec(32), _row_spec(32), _row_spec(32), _row_spec(32)),
        out_shape=outs,
    )(*u4, den, h, wa, ba, al, g1, b1, wq, bq, wkr, bkr, wvm, bvm)


def _last_call(u4, den, h, wa, ba, al, g1, b1):
    return pl.pallas_call(
        _last_body,
        grid=(NB,),
        in_specs=[_row_spec(32)] * 4 + [_DEN, _row_spec(128),
                  _W128, _B128, _SCAL, _B128, _B128],
        out_specs=pl.BlockSpec((1, 128), lambda i: (0, 0)),
        out_shape=jax.ShapeDtypeStruct((1, 128), _f32),
    )(*u4, den, h, wa, ba, al, g1, b1)


def _sc_mesh():
    return plsc.VectorSubcoreMesh(core_axis_name="c", subcore_axis_name="s",
                                  num_cores=_NC, num_subcores=_NS)


def _att_body(q_hbm, kr_hbm, src_hbm, dst_hbm, z1_hbm,
              ae_hbm, den0_hbm, den1_hbm,
              srcb, dstb, qrowsA, krrowsA, qrowsB, krrowsB,
              aebuf, den_sh, semA, semB):
    ci = lax.axis_index("c")
    si = lax.axis_index("s")
    t = ci * _NS + si
    e0 = t * 3200
    pltpu.sync_copy(z1_hbm, den_sh.at[pl.ds(si * _STR, _STR)])
    pltpu.sync_copy(src_hbm.at[pl.ds(e0, 3200)], srcb)

    def ldrow(j, carry):
        pltpu.sync_copy(dst_hbm.at[pl.ds(e0 + j * 128, 128)], dstb.at[j])
        return carry

    lax.fori_loop(0, 25, ldrow, 0)
    plsc.subcore_barrier()
    iota = lax.iota(_i32, 16)

    def start(j, qr, krr, sm):
        pltpu.async_copy(q_hbm.at[dstb.at[j]], qr, sm)
        pltpu.async_copy(kr_hbm.at[srcb.at[pl.ds(j * 128, 128)]], krr, sm)

    def wait(j, qr, krr, sm):
        pltpu.make_async_copy(q_hbm.at[dstb.at[j]], qr, sm).wait()
        pltpu.make_async_copy(kr_hbm.at[srcb.at[pl.ds(j * 128, 128)]],
                              krr, sm).wait()

    def compute(j, qr, krr):
        base = e0 + j * 128
        for g in range(8):
            rowv = g * 16 + iota

            def dstep(m, accs):
                a0, a1 = accs
                d0 = m * 8
                for u in range(8):
                    colv = jnp.zeros((16,), _i32) + (d0 + u)
                    qv = plsc.load_gather(qr, [rowv, colv])
                    kv = plsc.load_gather(krr, [rowv, colv])
                    if u % 2 == 0:
                        a0 = a0 + qv * kv
                    else:
                        a1 = a1 + qv * kv
                return a0, a1

            z16 = jnp.zeros((16,), _f32)
            a0, a1 = lax.fori_loop(0, 16, dstep, (z16, z16))
            acc = a0 + a1
            ae = jnp.exp(acc)
            gid = base + g * 16 + iota
            ae = jnp.where(gid < ER, ae, 0.0)
            aebuf[pl.ds(j * 128 + g * 16, 16)] = ae
        pltpu.sync_copy(aebuf.at[pl.ds(j * 128, 128)],
                        den_sh.at[dstb.at[j]], add=True)

    start(0, qrowsA, krrowsA, semA)

    def block2(k, carry):
        j = 2 * k
        start(j + 1, qrowsB, krrowsB, semB)
        wait(j, qrowsA, krrowsA, semA)
        compute(j, qrowsA, krrowsA)
        start(j + 2, qrowsA, krrowsA, semA)
        wait(j + 1, qrowsB, krrowsB, semB)
        compute(j + 1, qrowsB, krrowsB)
        return carry

    lax.fori_loop(0, 12, block2, 0)
    wait(24, qrowsA, krrowsA, semA)
    compute(24, qrowsA, krrowsA)
    pltpu.sync_copy(aebuf, ae_hbm.at[pl.ds(e0, 3200)])
    plsc.subcore_barrier()

    @pl.when(ci == 0)
    def _():
        pltpu.sync_copy(den_sh.at[pl.ds(si * _STR, _STR)],
                        den0_hbm.at[pl.ds(si * _STR, _STR)])

    @pl.when(ci == 1)
    def _():
        pltpu.sync_copy(den_sh.at[pl.ds(si * _STR, _STR)],
                        den1_hbm.at[pl.ds(si * _STR, _STR)])


def _att_call(q, kr, src3, dst3, z1):
    out_type = (jax.ShapeDtypeStruct((EP,), _f32),
                jax.ShapeDtypeStruct((NP,), _f32),
                jax.ShapeDtypeStruct((NP,), _f32))
    scratch = [
        pltpu.VMEM((3200,), _i32),
        pltpu.VMEM((25, 128), _i32),
        pltpu.VMEM((128, 128), _f32),
        pltpu.VMEM((128, 128), _f32),
        pltpu.VMEM((128, 128), _f32),
        pltpu.VMEM((128, 128), _f32),
        pltpu.VMEM((3200,), _f32),
        pltpu.VMEM_SHARED((NP,), _f32),
        pltpu.SemaphoreType.DMA,
        pltpu.SemaphoreType.DMA,
    ]
    f = functools.partial(
        pl.kernel, out_type=out_type, mesh=_sc_mesh(),
        scratch_types=scratch,
        compiler_params=pltpu.CompilerParams(needs_layout_passes=False),
    )(_att_body)
    return f(q, kr, src3, dst3, z1)


def _u_body(vm0, vm1, vm2, vm3, ae_hbm, src_hbm, dst_hbm, z2_hbm,
            u0, u1, u2, u3,
            srcb, dstb, aeb, vrowsA, vrowsB, u_sh, semA, semB):
    ci = lax.axis_index("c")
    si = lax.axis_index("s")
    e0 = si * 6400
    pltpu.sync_copy(src_hbm.at[pl.ds(e0, 6400)], srcb)
    pltpu.sync_copy(ae_hbm.at[pl.ds(e0, 6400)], aeb)

    def ldrow(j, carry):
        pltpu.sync_copy(dst_hbm.at[pl.ds(e0 + j * 128, 128)], dstb.at[j])
        return carry

    lax.fori_loop(0, 50, ldrow, 0)
    iota = lax.iota(_i32, 16)
    vms = (vm0, vm1, vm2, vm3)
    us = (u0, u1, u2, u3)
    for p in range(2):
        pltpu.sync_copy(z2_hbm, u_sh.at[pl.ds(si * _STR, _STR)])
        plsc.subcore_barrier()
        for cival in range(2):
            chunk = 2 * p + cival

            @pl.when(ci == cival)
            def _(chunk=chunk):
                vmr = vms[chunk]

                def start(j, vr, sm):
                    pltpu.async_copy(vmr.at[srcb.at[pl.ds(j * 128, 128)]],
                                     vr, sm)

                def compute(j, vr, sm):
                    pltpu.make_async_copy(
                        vmr.at[srcb.at[pl.ds(j * 128, 128)]], vr, sm).wait()
                    for g in range(8):
                        rowv = g * 16 + iota
                        aev = aeb[pl.ds(j * 128 + g * 16, 16)]

                        def dstep(m, cc):
                            d0 = m * 8
                            for u in range(8):
                                colv = jnp.zeros((16,), _i32) + (d0 + u)
                                x = plsc.load_gather(vr, [rowv, colv])
                                plsc.store_scatter(vr, [rowv, colv],
                                                   x * aev)
                            return cc

                        lax.fori_loop(0, 4, dstep, 0)
                    pltpu.sync_copy(vr, u_sh.at[dstb.at[j]], add=True)

                start(0, vrowsA, semA)

                def block2(k, carry):
                    j = 2 * k
                    start(j + 1, vrowsB, semB)
                    compute(j, vrowsA, semA)

                    @pl.when(j + 2 < 50)
                    def _():
                        start(j + 2, vrowsA, semA)

                    compute(j + 1, vrowsB, semB)
                    return carry

                lax.fori_loop(0, 25, block2, 0)
        plsc.subcore_barrier()
        for cival in range(2):
            chunk = 2 * p + cival

            @pl.when(ci == cival)
            def _(chunk=chunk):
                pltpu.sync_copy(u_sh.at[pl.ds(si * _STR, _STR)],
                                us[chunk].at[pl.ds(si * _STR, _STR)])
        plsc.subcore_barrier()


def _u_call(vm4, ae3u, src3u, dst3u, z2):
    out_type = tuple(jax.ShapeDtypeStruct((NP, 32), _f32) for _ in range(4))
    scratch = [
        pltpu.VMEM((6400,), _i32),
        pltpu.VMEM((50, 128), _i32),
        pltpu.VMEM((6400,), _f32),
        pltpu.VMEM((128, 32), _f32),
        pltpu.VMEM((128, 32), _f32),
        pltpu.VMEM_SHARED((NP, 32), _f32),
        pltpu.SemaphoreType.DMA,
        pltpu.SemaphoreType.DMA,
    ]
    f = functools.partial(
        pl.kernel, out_type=out_type, mesh=_sc_mesh(),
        scratch_types=scratch,
        compiler_params=pltpu.CompilerParams(needs_layout_passes=False,
                                             use_tc_tiling_on_sc=False),
    )(_u_body)
    return f(*vm4, ae3u, src3u, dst3u, z2)


def kernel(x, edge_index, edge_weight, params):
    lys = params['layers']
    n_layers = len(lys)
    conv_w = []
    for l in range(n_layers):
        lp = lys[l]
        for i in range(NR):
            s = lp['pri'][i, 0] / np.sqrt(np.float32(HID))
            conv_w.append(dict(
                wq=lp['Wq'], bq=lp['bq'][None, :],
                wkr=(lp['Wk'] @ lp['ratt'][i, 0]) * s,
                bkr=(lp['bk'] @ lp['ratt'][i, 0])[None, :] * s,
                wvm=lp['Wv'] @ lp['rmsg'][i, 0],
                bvm=(lp['bv'] @ lp['rmsg'][i, 0])[None, :],
                wa=lp['Wa'], ba=lp['ba'][None, :],
                al=jax.nn.sigmoid(lp['skip']).reshape(1, 1),
                g1=lp['g1'][None, :], b1=lp['b1'][None, :]))

    srcs, dsts = [], []
    for i in range(NR):
        srcs.append(jnp.pad(edge_index[i, 0], (0, EP - ER)))
        dsts.append(jnp.pad(edge_index[i, 1], (0, EP - ER)))
    z1 = jnp.zeros((_STR,), _f32)
    z2 = jnp.zeros((_STR, 32), _f32)

    x_p = jnp.pad(x, ((0, NP - NN), (0, 0)))
    w0 = conv_w[0]
    h, q, kr, *vm4 = _d0_call(
        x_p, params['Wp'], params['bp'][None, :],
        w0['wq'], w0['bq'], w0['wkr'], w0['bkr'], w0['wvm'], w0['bvm'])
    vm4 = tuple(vm4)

    n_conv = n_layers * NR
    for j in range(n_conv):
        ae1, den0, den1 = _att_call(q, kr, srcs[j % NR], dsts[j % NR], z1)
        u4 = _u_call(vm4, ae1, srcs[j % NR], dsts[j % NR], z2)
        den_t = jnp.stack([den0, den1], axis=1)
        w = conv_w[j]
        if j + 1 < n_conv:
            wn = conv_w[j + 1]
            h, q, kr, *vm4 = _mid_call(
                u4, den_t, h, w['wa'], w['ba'], w['al'], w['g1'], w['b1'],
                wn['wq'], wn['bq'], wn['wkr'], wn['bkr'], wn['wvm'], wn['bvm'])
            vm4 = tuple(vm4)
        else:
            hsum = _last_call(u4, den_t, h, w['wa'], w['ba'], w['al'],
                              w['g1'], w['b1'])

    hg = hsum[0] / np.float32(NN)
    logits = (hg @ params['Wc'] + params['bc']).squeeze(-1)
    return logits
